# baseline (speedup 1.0000x reference)
import functools

import jax
import jax.numpy as jnp
from jax import lax
from jax.experimental import pallas as pl
from jax.experimental.pallas import tpu as pltpu
from jax.experimental.pallas import tpu_sc as plsc

B, I, K, W = 4, 2048, 16, 32
N = B * I * K
NW = 32
PER_W = N // NW
GRP = PER_W // 16
BI_PER_W = PER_W // K
W_PER_B = (B * I) // BI_PER_W // B


def _sc_partials(ll1, la1, pp1, pn1, ind1, m1, il16):
    mesh = plsc.VectorSubcoreMesh(core_axis_name="c", subcore_axis_name="s")

    @functools.partial(
        pl.kernel,
        mesh=mesh,
        compiler_params=pltpu.CompilerParams(needs_layout_passes=False),
        out_type=jax.ShapeDtypeStruct((NW, 5, 16), jnp.float32),
        scratch_types=[
            pltpu.VMEM((PER_W,), jnp.int32),
            pltpu.VMEM((BI_PER_W,), jnp.int32),
            pltpu.VMEM((16,), jnp.int32),
            pltpu.VMEM((PER_W,), jnp.int32),
            pltpu.VMEM((PER_W,), jnp.int32),
            pltpu.VMEM((PER_W,), jnp.float32),
            pltpu.VMEM((PER_W,), jnp.float32),
            pltpu.VMEM((PER_W,), jnp.float32),
            pltpu.VMEM((PER_W,), jnp.float32),
            pltpu.VMEM((5, 16), jnp.float32),
            pltpu.SemaphoreType.DMA,
        ],
    )
    def body(ll_h, la_h, pp_h, pn_h, ind_h, m_h, il_h, out_h,
             m_v, ind_v, il_v, idxa, idxb, llg, lag, ppg, png, acc_v, sem):
        cid = lax.axis_index("c")
        sid = lax.axis_index("s")
        w = sid * 2 + cid

        pltpu.sync_copy(m_h.at[pl.ds(w * PER_W, PER_W)], m_v)
        pltpu.sync_copy(ind_h.at[pl.ds(w * BI_PER_W, BI_PER_W)], ind_v)
        pltpu.sync_copy(il_h, il_v)

        lane = lax.iota(jnp.int32, 16)
        lane_w = lane * W

        def build(t, carry):
            m = m_v[pl.ds(t * 16, 16)]
            s = plsc.load_gather(ind_v, [jnp.full((16,), t, jnp.int32)])
            im = jnp.maximum(m - s - 1, 0)
            base = (w * PER_W + t * 16) * W
            idxa[pl.ds(t * 16, 16)] = base + lane_w + im
            idxb[pl.ds(t * 16, 16)] = base + lane_w
            return carry

        lax.fori_loop(0, GRP, build, 0)

        def gather_chunk(c, carry):
            sl = pl.ds(c * 128, 128)
            pltpu.async_copy(ll_h.at[idxa.at[sl]], llg.at[sl], sem)
            pltpu.async_copy(la_h.at[idxa.at[sl]], lag.at[sl], sem)
            pltpu.async_copy(pp_h.at[idxa.at[sl]], ppg.at[sl], sem)
            pltpu.async_copy(pn_h.at[idxb.at[sl]], png.at[sl], sem)
            pltpu.make_async_copy(ll_h.at[idxa.at[sl]], llg.at[sl], sem).wait()
            pltpu.make_async_copy(la_h.at[idxa.at[sl]], lag.at[sl], sem).wait()
            pltpu.make_async_copy(pp_h.at[idxa.at[sl]], ppg.at[sl], sem).wait()
            pltpu.make_async_copy(pn_h.at[idxb.at[sl]], png.at[sl], sem).wait()
            return carry

        lax.fori_loop(0, PER_W // 128, gather_chunk, 0)

        ilb = plsc.load_gather(
            il_v, [jnp.full((16,), w // W_PER_B, jnp.int32)])
        i_base = (w % W_PER_B) * BI_PER_W
        zero = jnp.zeros((16,), jnp.float32)
        one = jnp.float32(1.0)
        fzero = jnp.float32(0.0)

        def accum(t, carry):
            a_ll, a_la, a_p, a_cm, a_ci = carry
            m = m_v[pl.ds(t * 16, 16)]
            mm = m >= 0
            mmf = jnp.where(mm, one, fzero)
            c0 = t * 16
            llv = llg[pl.ds(c0, 16)]
            lav = lag[pl.ds(c0, 16)]
            ppv = ppg[pl.ds(c0, 16)]
            pnv = png[pl.ds(c0, 16)]
            pres = jnp.where(mm, ppv, -pnv)
            imf = jnp.where(i_base + t < ilb, one, fzero)
            return (a_ll + llv * mmf, a_la + lav * mmf,
                    a_p + pres * imf, a_cm + mmf, a_ci + imf)

        a_ll, a_la, a_p, a_cm, a_ci = lax.fori_loop(
            0, GRP, accum, (zero, zero, zero, zero, zero))

        acc_v[0] = a_ll
        acc_v[1] = a_la
        acc_v[2] = a_p
        acc_v[3] = a_cm
        acc_v[4] = a_ci
        pltpu.sync_copy(acc_v, out_h.at[w])

    return body(ll1, la1, pp1, pn1, ind1, m1, il16)


def _tc_combine(partials):
    def body(p_ref, o_ref):
        x = p_ref[...]
        s_ll = jnp.sum(x[:, 0, :])
        s_la = jnp.sum(x[:, 1, :])
        s_p = jnp.sum(x[:, 2, :])
        s_cm = jnp.sum(x[:, 3, :])
        s_ci = jnp.sum(x[:, 4, :])
        iot = lax.broadcasted_iota(jnp.int32, (1, 128), 1)
        o_ref[...] = (jnp.where(iot == 0, s_ll / s_cm, 0.0)
                      + jnp.where(iot == 1, s_la / s_cm, 0.0)
                      + jnp.where(iot == 2, s_p / s_ci, 0.0))

    return pl.pallas_call(
        body, out_shape=jax.ShapeDtypeStruct((1, 128), jnp.float32))(partials)


def kernel(loss_labels, loss_amounts, presence_pos, presence_neg,
           indices, matching, idx_lens):
    ll1 = loss_labels.reshape(-1)
    la1 = loss_amounts.reshape(-1)
    pp1 = presence_pos.reshape(-1)
    pn1 = presence_neg.reshape(-1)
    ind1 = indices.reshape(-1)
    m1 = matching.reshape(-1)
    il16 = jnp.pad(idx_lens, (0, 16 - idx_lens.shape[0]))
    partials = _sc_partials(ll1, la1, pp1, pn1, ind1, m1, il16)
    out = _tc_combine(partials)
    return (out[0, 0], out[0, 1], out[0, 2])


# speedup vs baseline: 1.0675x; 1.0675x over previous
import functools

import jax
import jax.numpy as jnp
from jax import lax
from jax.experimental import pallas as pl
from jax.experimental.pallas import tpu as pltpu
from jax.experimental.pallas import tpu_sc as plsc

B, I, K, W = 4, 2048, 16, 32
N = B * I * K
NW = 32
PER_W = N // NW
GRP = PER_W // 16
BI_PER_W = PER_W // K
W_PER_B = (B * I) // BI_PER_W // B


def _sc_partials(ll1, la1, pp1, pn1, ind1, m1, il16):
    mesh = plsc.VectorSubcoreMesh(core_axis_name="c", subcore_axis_name="s")

    @functools.partial(
        pl.kernel,
        mesh=mesh,
        compiler_params=pltpu.CompilerParams(needs_layout_passes=False),
        out_type=jax.ShapeDtypeStruct((NW, 5, 16), jnp.float32),
        scratch_types=[
            pltpu.VMEM((PER_W,), jnp.int32),
            pltpu.VMEM((BI_PER_W,), jnp.int32),
            pltpu.VMEM((16,), jnp.int32),
            pltpu.VMEM((PER_W,), jnp.int32),
            pltpu.VMEM((PER_W,), jnp.float32),
            pltpu.VMEM((PER_W,), jnp.float32),
            pltpu.VMEM((PER_W,), jnp.float32),
            pltpu.VMEM((PER_W,), jnp.float32),
            pltpu.VMEM((5, 16), jnp.float32),
            pltpu.SemaphoreType.DMA,
        ],
    )
    def body(ll_h, la_h, pp_h, pn_h, ind_h, m_h, il_h, out_h,
             m_v, ind_v, il_v, idxa, llg, lag, ppg, png, acc_v, sem):
        cid = lax.axis_index("c")
        sid = lax.axis_index("s")
        w = sid * 2 + cid

        pltpu.sync_copy(m_h.at[pl.ds(w * PER_W, PER_W)], m_v)
        pltpu.sync_copy(ind_h.at[pl.ds(w * BI_PER_W, BI_PER_W)], ind_v)
        pltpu.sync_copy(il_h, il_v)

        lane = lax.iota(jnp.int32, 16)
        lane_w = lane * W

        def build(t, carry):
            m = m_v[pl.ds(t * 16, 16)]
            s = plsc.load_gather(ind_v, [jnp.full((16,), t, jnp.int32)])
            im = jnp.maximum(m - s - 1, 0)
            base = (w * PER_W + t * 16) * W
            idxa[pl.ds(t * 16, 16)] = base + lane_w + im
            return carry

        lax.fori_loop(0, GRP, build, 0)

        pltpu.async_copy(ll_h.at[idxa], llg, sem)
        pltpu.async_copy(la_h.at[idxa], lag, sem)
        pltpu.async_copy(pp_h.at[idxa], ppg, sem)
        pltpu.async_copy(pn_h.at[idxa], png, sem)
        pltpu.make_async_copy(ll_h.at[idxa], llg, sem).wait()
        pltpu.make_async_copy(la_h.at[idxa], lag, sem).wait()
        pltpu.make_async_copy(pp_h.at[idxa], ppg, sem).wait()
        pltpu.make_async_copy(pn_h.at[idxa], png, sem).wait()

        ilb = plsc.load_gather(
            il_v, [jnp.full((16,), w // W_PER_B, jnp.int32)])
        i_base = (w % W_PER_B) * BI_PER_W
        zero = jnp.zeros((16,), jnp.float32)
        one = jnp.float32(1.0)
        fzero = jnp.float32(0.0)

        def accum(t, carry):
            a_ll, a_la, a_p, a_cm, a_ci = carry
            m = m_v[pl.ds(t * 16, 16)]
            mm = m >= 0
            mmf = jnp.where(mm, one, fzero)
            c0 = t * 16
            llv = llg[pl.ds(c0, 16)]
            lav = lag[pl.ds(c0, 16)]
            ppv = ppg[pl.ds(c0, 16)]
            pnv = png[pl.ds(c0, 16)]
            pres = jnp.where(mm, ppv, -pnv)
            imf = jnp.where(i_base + t < ilb, one, fzero)
            return (a_ll + llv * mmf, a_la + lav * mmf,
                    a_p + pres * imf, a_cm + mmf, a_ci + imf)

        a_ll, a_la, a_p, a_cm, a_ci = lax.fori_loop(
            0, GRP, accum, (zero, zero, zero, zero, zero))

        acc_v[0] = a_ll
        acc_v[1] = a_la
        acc_v[2] = a_p
        acc_v[3] = a_cm
        acc_v[4] = a_ci
        pltpu.sync_copy(acc_v, out_h.at[w])

    return body(ll1, la1, pp1, pn1, ind1, m1, il16)


def _tc_combine(partials):
    def body(p_ref, o_ref):
        x = p_ref[...]
        s_ll = jnp.sum(x[:, 0, :])
        s_la = jnp.sum(x[:, 1, :])
        s_p = jnp.sum(x[:, 2, :])
        s_cm = jnp.sum(x[:, 3, :])
        s_ci = jnp.sum(x[:, 4, :])
        iot = lax.broadcasted_iota(jnp.int32, (1, 128), 1)
        o_ref[...] = (jnp.where(iot == 0, s_ll / s_cm, 0.0)
                      + jnp.where(iot == 1, s_la / s_cm, 0.0)
                      + jnp.where(iot == 2, s_p / s_ci, 0.0))

    return pl.pallas_call(
        body, out_shape=jax.ShapeDtypeStruct((1, 128), jnp.float32))(partials)


def kernel(loss_labels, loss_amounts, presence_pos, presence_neg,
           indices, matching, idx_lens):
    ll1 = loss_labels.reshape(-1)
    la1 = loss_amounts.reshape(-1)
    pp1 = presence_pos.reshape(-1)
    pn1 = presence_neg.reshape(-1)
    ind1 = indices.reshape(-1)
    m1 = matching.reshape(-1)
    il16 = jnp.pad(idx_lens, (0, 16 - idx_lens.shape[0]))
    partials = _sc_partials(ll1, la1, pp1, pn1, ind1, m1, il16)
    out = _tc_combine(partials)
    return (out[0, 0], out[0, 1], out[0, 2])


# speedup vs baseline: 4.1595x; 3.8963x over previous
import jax
import jax.numpy as jnp
from jax import lax
from jax.experimental import pallas as pl
from jax.experimental.pallas import tpu as pltpu

B, I, K, W = 4, 2048, 16, 32


def _body(il_ref, ll_ref, la_ref, pp_ref, pn_ref, m_ref, ind_ref, o_ref):
    b = pl.program_id(0)
    k = pl.program_id(1)

    @pl.when(jnp.logical_and(b == 0, k == 0))
    def _init():
        o_ref[...] = jnp.zeros_like(o_ref)

    mm = m_ref[0, 0]
    ind = ind_ref[0]
    im = jnp.maximum(mm - ind - 1, 0)
    mmff = jnp.where(mm >= 0, 1.0, 0.0).astype(jnp.float32)

    il = il_ref[b]
    iota_i = lax.broadcasted_iota(jnp.int32, (1, I), 1)
    imff = jnp.where(iota_i < il, 1.0, 0.0).astype(jnp.float32)

    wiota = lax.broadcasted_iota(jnp.int32, (W, I), 0)
    wself = jnp.where(wiota == im, 1.0, 0.0).astype(jnp.float32)
    m1f = wself * mmff
    ppm = m1f * imff
    pnm = (wself - m1f) * imff

    s_ll = jnp.sum(ll_ref[0, 0] * m1f)
    s_la = jnp.sum(la_ref[0, 0] * m1f)
    s_p = jnp.sum(pp_ref[0, 0] * ppm) - jnp.sum(pn_ref[0, 0] * pnm)
    s_cm = jnp.sum(mmff) * jnp.float32(1.0)

    lane = lax.broadcasted_iota(jnp.int32, (1, 128), 1)
    o_ref[...] += (jnp.where(lane == 0, s_ll, 0.0)
                   + jnp.where(lane == 1, s_la, 0.0)
                   + jnp.where(lane == 2, s_p, 0.0)
                   + jnp.where(lane == 3, s_cm, 0.0))


def kernel(loss_labels, loss_amounts, presence_pos, presence_neg,
           indices, matching, idx_lens):
    llt = jnp.transpose(loss_labels, (0, 2, 3, 1))
    lat = jnp.transpose(loss_amounts, (0, 2, 3, 1))
    ppt = jnp.transpose(presence_pos, (0, 2, 3, 1))
    pnt = jnp.transpose(presence_neg, (0, 2, 3, 1))
    mt = jnp.transpose(matching, (0, 2, 1)).reshape(B, K, 1, I)
    ind3 = indices.reshape(B, 1, I)

    grid_spec = pltpu.PrefetchScalarGridSpec(
        num_scalar_prefetch=1,
        grid=(B, K),
        in_specs=[
            pl.BlockSpec((1, 1, W, I), lambda b, k, il: (b, k, 0, 0)),
            pl.BlockSpec((1, 1, W, I), lambda b, k, il: (b, k, 0, 0)),
            pl.BlockSpec((1, 1, W, I), lambda b, k, il: (b, k, 0, 0)),
            pl.BlockSpec((1, 1, W, I), lambda b, k, il: (b, k, 0, 0)),
            pl.BlockSpec((1, 1, 1, I), lambda b, k, il: (b, k, 0, 0)),
            pl.BlockSpec((1, 1, I), lambda b, k, il: (b, 0, 0)),
        ],
        out_specs=pl.BlockSpec((1, 128), lambda b, k, il: (0, 0)),
    )
    out = pl.pallas_call(
        _body,
        grid_spec=grid_spec,
        out_shape=jax.ShapeDtypeStruct((1, 128), jnp.float32),
    )(idx_lens, llt, lat, ppt, pnt, mt, ind3)

    denom_p = jnp.sum(idx_lens).astype(jnp.float32) * jnp.float32(K)
    return (out[0, 0] / out[0, 3], out[0, 1] / out[0, 3], out[0, 2] / denom_p)
